# Initial kernel scaffold; baseline (speedup 1.0000x reference)
import jax
import jax.numpy as jnp
from jax.experimental import pallas as pl


def kernel(label_ids, weight):
    raise NotImplementedError("write your pallas kernel here")


# speedup vs baseline: 6.4508x; 6.4508x over previous
import functools

import jax
import jax.numpy as jnp
from jax import lax
from jax.experimental import pallas as pl
from jax.experimental.pallas import tpu as pltpu
from jax.experimental.pallas import tpu_sc as plsc

_NUM_WORKERS = 32
_CHUNK = 512


def _sc_gather(idx_flat, weight):
    n = idx_flat.shape[0]
    d = weight.shape[1]
    per_w = n // _NUM_WORKERS
    n_chunks = per_w // _CHUNK

    mesh = plsc.VectorSubcoreMesh(core_axis_name="c", subcore_axis_name="s")

    @functools.partial(
        pl.kernel,
        out_type=jax.ShapeDtypeStruct((n, d), jnp.float32),
        mesh=mesh,
        scratch_types=[
            pltpu.VMEM((_CHUNK,), jnp.int32),
            pltpu.VMEM((_CHUNK, d), jnp.float32),
            pltpu.SemaphoreType.DMA,
        ],
    )
    def k(idx_hbm, table_hbm, out_hbm, idx_v, rows_v, sem):
        wid = lax.axis_index("s") * 2 + lax.axis_index("c")
        base = wid * per_w

        def body(g, carry):
            off = pl.multiple_of(base + g * _CHUNK, 8)
            pltpu.sync_copy(idx_hbm.at[pl.ds(off, _CHUNK)], idx_v)
            pltpu.async_copy(table_hbm.at[idx_v], rows_v, sem).wait()
            pltpu.sync_copy(rows_v, out_hbm.at[pl.ds(off, _CHUNK)])
            return carry

        lax.fori_loop(0, n_chunks, body, 0)

    return k(idx_flat, weight)


def kernel(label_ids, weight):
    b, l = label_ids.shape
    d = weight.shape[1]
    idx_flat = label_ids.reshape(-1).astype(jnp.int32)
    out = _sc_gather(idx_flat, weight)
    return out.reshape(b, l, d)


# speedup vs baseline: 6.5318x; 1.0126x over previous
import functools

import jax
import jax.numpy as jnp
from jax import lax
from jax.experimental import pallas as pl
from jax.experimental.pallas import tpu as pltpu
from jax.experimental.pallas import tpu_sc as plsc

_NUM_WORKERS = 32
_CHUNK = 400


def _sc_gather(idx_flat, weight):
    n = idx_flat.shape[0]
    d = weight.shape[1]
    per_w = n // _NUM_WORKERS
    n_chunks = per_w // _CHUNK
    assert n_chunks % 2 == 0 and n_chunks >= 4

    mesh = plsc.VectorSubcoreMesh(core_axis_name="c", subcore_axis_name="s")

    @functools.partial(
        pl.kernel,
        out_type=jax.ShapeDtypeStruct((n, d), jnp.float32),
        mesh=mesh,
        scratch_types=[
            pltpu.VMEM((per_w,), jnp.int32),
            pltpu.VMEM((2, _CHUNK, d), jnp.float32),
            pltpu.SemaphoreType.DMA,
            pltpu.SemaphoreType.DMA,
            pltpu.SemaphoreType.DMA,
        ],
    )
    def k(idx_hbm, table_hbm, out_hbm, idx_v, rows_v, gsem, osem0, osem1):
        wid = lax.axis_index("s") * 2 + lax.axis_index("c")
        base = pl.multiple_of(wid * per_w, 8)
        pltpu.sync_copy(idx_hbm.at[pl.ds(base, per_w)], idx_v)
        osems = (osem0, osem1)

        def gather(g, b):
            off = pl.multiple_of(g * _CHUNK, 8)
            pltpu.async_copy(
                table_hbm.at[idx_v.at[pl.ds(off, _CHUNK)]], rows_v.at[b], gsem
            ).wait()

        def scatter_start(g, b):
            off = pl.multiple_of(base + g * _CHUNK, 8)
            pltpu.async_copy(
                rows_v.at[b], out_hbm.at[pl.ds(off, _CHUNK)], osems[b]
            )

        def scatter_wait(b):
            pltpu.make_async_copy(
                rows_v.at[b], out_hbm.at[pl.ds(base, _CHUNK)], osems[b]
            ).wait()

        for b in range(2):
            gather(b, b)
            scatter_start(b, b)

        def body(i, carry):
            t = 2 * i
            for b in range(2):
                g = t + b
                scatter_wait(b)
                gather(g, b)
                scatter_start(g, b)
            return carry

        lax.fori_loop(1, n_chunks // 2, body, 0)

        for b in range(2):
            scatter_wait(b)

    return k(idx_flat, weight)


def kernel(label_ids, weight):
    b, l = label_ids.shape
    d = weight.shape[1]
    idx_flat = label_ids.reshape(-1).astype(jnp.int32)
    out = _sc_gather(idx_flat, weight)
    return out.reshape(b, l, d)


# speedup vs baseline: 15.4282x; 2.3620x over previous
import functools

import jax
import jax.numpy as jnp
from jax import lax
from jax.experimental import pallas as pl
from jax.experimental.pallas import tpu as pltpu
from jax.experimental.pallas import tpu_sc as plsc

_NUM_WORKERS = 32
_CHUNK = 320


def _sc_gather(idx_flat, weight):
    n = idx_flat.shape[0]
    d = weight.shape[1]
    per_w = n // _NUM_WORKERS
    n_chunks = per_w // _CHUNK
    assert n_chunks % 2 == 0 and n_chunks >= 4

    mesh = plsc.VectorSubcoreMesh(core_axis_name="c", subcore_axis_name="s")

    @functools.partial(
        pl.kernel,
        out_type=jax.ShapeDtypeStruct((n, d), jnp.float32),
        mesh=mesh,
        scratch_types=[
            pltpu.VMEM_SHARED(weight.shape, jnp.float32),
            pltpu.VMEM((per_w,), jnp.int32),
            pltpu.VMEM((2, _CHUNK, d), jnp.float32),
            pltpu.SemaphoreType.DMA,
            pltpu.SemaphoreType.DMA,
            pltpu.SemaphoreType.DMA,
        ],
    )
    def k(idx_hbm, table_hbm, out_hbm, table_sh, idx_v, rows_v, gsem, osem0, osem1):
        sid = lax.axis_index("s")
        wid = sid * 2 + lax.axis_index("c")
        base = pl.multiple_of(wid * per_w, 8)

        @pl.when(sid == 0)
        def _():
            pltpu.sync_copy(table_hbm, table_sh)

        pltpu.sync_copy(idx_hbm.at[pl.ds(base, per_w)], idx_v)
        plsc.subcore_barrier()
        osems = (osem0, osem1)

        def gather(g, b):
            off = pl.multiple_of(g * _CHUNK, 8)
            pltpu.async_copy(
                table_sh.at[idx_v.at[pl.ds(off, _CHUNK)]], rows_v.at[b], gsem
            ).wait()

        def scatter_start(g, b):
            off = pl.multiple_of(base + g * _CHUNK, 8)
            pltpu.async_copy(
                rows_v.at[b], out_hbm.at[pl.ds(off, _CHUNK)], osems[b]
            )

        def scatter_wait(b):
            pltpu.make_async_copy(
                rows_v.at[b], out_hbm.at[pl.ds(base, _CHUNK)], osems[b]
            ).wait()

        for b in range(2):
            gather(b, b)
            scatter_start(b, b)

        def body(i, carry):
            t = 2 * i
            for b in range(2):
                g = t + b
                scatter_wait(b)
                gather(g, b)
                scatter_start(g, b)
            return carry

        lax.fori_loop(1, n_chunks // 2, body, 0)

        for b in range(2):
            scatter_wait(b)

    return k(idx_flat, weight)


def kernel(label_ids, weight):
    b, l = label_ids.shape
    d = weight.shape[1]
    idx_flat = label_ids.reshape(-1).astype(jnp.int32)
    out = _sc_gather(idx_flat, weight)
    return out.reshape(b, l, d)
